# Initial kernel scaffold; baseline (speedup 1.0000x reference)
import jax
import jax.numpy as jnp
from jax.experimental import pallas as pl


def kernel(positions, atomic_numbers, neighbors, emb, W_f1, W_msg, b_msg, W_o1, b_o1, W_o2, b_o2):
    raise NotImplementedError("write your pallas kernel here")


# speedup vs baseline: 6.2464x; 6.2464x over previous
import jax
import jax.numpy as jnp
import numpy as np
from jax.experimental import pallas as pl
from jax.experimental.pallas import tpu as pltpu

_B, _A, _N, _D, _NRBF, _ZMAX, _H = 8, 1024, 48, 128, 32, 100, 64
_TA = 64
_NT = _A // _TA
_P = _TA * _N


def _body(nbr_ref, ohc_ref, pos_ref, z_ref, emb_ref, wf1_ref, wmsg_ref,
          bmsg_ref, wo1_ref, bo1_ref, wo2_ref, out_ref,
          x_f32, x_bf16, posq_hi, posq_lo):
    t = pl.program_id(1)

    @pl.when(t == 0)
    def _init():
        z = z_ref[0, 0]
        ohz = (z[:, None] == jax.lax.broadcasted_iota(
            jnp.int32, (_A, _ZMAX), 1)).astype(jnp.float32)
        xb = jnp.dot(ohz, emb_ref[...],
                     preferred_element_type=jnp.float32)
        x_f32[...] = xb
        x_bf16[...] = xb.astype(jnp.bfloat16)
        p = pos_ref[0]
        pq = jnp.concatenate([p, jnp.zeros((_A, 5), jnp.float32)], axis=1)
        hi = pq.astype(jnp.bfloat16)
        posq_hi[...] = hi
        posq_lo[...] = (pq - hi.astype(jnp.float32)).astype(jnp.bfloat16)

    nbr = nbr_ref[0]
    oh = (nbr == jax.lax.broadcasted_iota(
        jnp.int32, (_P, _A), 1)).astype(jnp.bfloat16)

    r_j = (jnp.dot(oh, posq_hi[...], preferred_element_type=jnp.float32)
           + jnp.dot(oh, posq_lo[...], preferred_element_type=jnp.float32))
    ohc = ohc_ref[...]
    hi_t = posq_hi[pl.ds(t * _TA, _TA), :].astype(jnp.float32)
    lo_t = posq_lo[pl.ds(t * _TA, _TA), :].astype(jnp.float32)
    r_i = jnp.dot(ohc, hi_t + lo_t,
                  preferred_element_type=jnp.float32)
    diff = r_j - r_i
    d2 = jnp.sum(diff * diff, axis=1, keepdims=True)
    d = jnp.sqrt(d2 + 1e-8)

    centers = jax.lax.broadcasted_iota(
        jnp.int32, (1, _NRBF), 1).astype(jnp.float32) * (5.0 / (_NRBF - 1))
    delta = d - centers
    rbf = jnp.exp(-10.0 * delta * delta)
    wij = jnp.dot(rbf, wf1_ref[...],
                  preferred_element_type=jnp.float32)

    x_j = jnp.dot(oh, x_bf16[...],
                  preferred_element_type=jnp.float32)
    prod = x_j * wij
    msg = jax.lax.dot_general(ohc, prod, (((0,), (0,)), ((), ())),
                              preferred_element_type=jnp.float32)

    rep = (x_f32[pl.ds(t * _TA, _TA), :]
           + jnp.tanh(jnp.dot(msg, wmsg_ref[...],
                              preferred_element_type=jnp.float32)
                      + bmsg_ref[...]))
    h1 = jnp.tanh(jnp.dot(rep, wo1_ref[...],
                          preferred_element_type=jnp.float32)
                  + bo1_ref[...])
    atom_e = jnp.dot(h1, wo2_ref[...],
                     preferred_element_type=jnp.float32)
    te = jnp.sum(atom_e)

    @pl.when(t == 0)
    def _first():
        out_ref[...] = jnp.full((1, 1, 128), te, jnp.float32)

    @pl.when(t != 0)
    def _acc():
        out_ref[...] += jnp.full((1, 1, 128), te, jnp.float32)


def kernel(positions, atomic_numbers, neighbors, emb, W_f1, W_msg, b_msg,
           W_o1, b_o1, W_o2, b_o2):
    z3 = atomic_numbers.astype(jnp.int32).reshape(_B, 1, _A)
    nbr = neighbors.astype(jnp.int32).reshape(_B * _NT, _P, 1)
    ohc = jnp.asarray(np.kron(np.eye(_TA, dtype=np.float32),
                              np.ones((_N, 1), np.float32)))
    grid = (_B, _NT)
    out = pl.pallas_call(
        _body,
        grid=grid,
        in_specs=[
            pl.BlockSpec((1, _P, 1), lambda b, t: (b * _NT + t, 0, 0)),
            pl.BlockSpec((_P, _TA), lambda b, t: (0, 0)),
            pl.BlockSpec((1, _A, 3), lambda b, t: (b, 0, 0)),
            pl.BlockSpec((1, 1, _A), lambda b, t: (b, 0, 0)),
            pl.BlockSpec((_ZMAX, _D), lambda b, t: (0, 0)),
            pl.BlockSpec((_NRBF, _D), lambda b, t: (0, 0)),
            pl.BlockSpec((_D, _D), lambda b, t: (0, 0)),
            pl.BlockSpec((1, _D), lambda b, t: (0, 0)),
            pl.BlockSpec((_D, _H), lambda b, t: (0, 0)),
            pl.BlockSpec((1, _H), lambda b, t: (0, 0)),
            pl.BlockSpec((_H, 1), lambda b, t: (0, 0)),
        ],
        out_specs=pl.BlockSpec((1, 1, 128), lambda b, t: (b, 0, 0)),
        out_shape=jax.ShapeDtypeStruct((_B, 1, 128), jnp.float32),
        scratch_shapes=[
            pltpu.VMEM((_A, _D), jnp.float32),
            pltpu.VMEM((_A, _D), jnp.bfloat16),
            pltpu.VMEM((_A, 8), jnp.bfloat16),
            pltpu.VMEM((_A, 8), jnp.bfloat16),
        ],
    )(nbr, ohc, positions, z3, emb, W_f1, W_msg, b_msg.reshape(1, _D),
      W_o1, b_o1.reshape(1, _H), W_o2)
    return out[:, 0, :1] + _A * b_o2[0]


# speedup vs baseline: 7.8614x; 1.2586x over previous
import jax
import jax.numpy as jnp
import numpy as np
from jax.experimental import pallas as pl
from jax.experimental.pallas import tpu as pltpu

_B, _A, _N, _D, _NRBF, _ZMAX, _H = 8, 1024, 48, 128, 32, 100, 64
_TA = 128
_NT = _A // _TA
_P = _TA * _N
_TW = _D + 16


def _body(nbr_ref, ohc_ref, pos_ref, z_ref, emb_ref, wf1_ref, wmsg_ref,
          bmsg_ref, wo1_ref, bo1_ref, wo2_ref, out_ref,
          x_f32, tbl):
    t = pl.program_id(1)

    @pl.when(t == 0)
    def _init():
        z = z_ref[0, 0]
        ohz = (z[:, None] == jax.lax.broadcasted_iota(
            jnp.int32, (_A, _ZMAX), 1)).astype(jnp.float32)
        xb = jnp.dot(ohz, emb_ref[...],
                     preferred_element_type=jnp.float32)
        x_f32[...] = xb
        p = pos_ref[0]
        pq = jnp.concatenate([p, jnp.zeros((_A, 5), jnp.float32)], axis=1)
        hi = pq.astype(jnp.bfloat16)
        lo = (pq - hi.astype(jnp.float32)).astype(jnp.bfloat16)
        tbl[...] = jnp.concatenate([xb.astype(jnp.bfloat16), hi, lo], axis=1)

    nbr16 = nbr_ref[0].astype(jnp.int16)
    oh = (nbr16 == jax.lax.broadcasted_iota(
        jnp.int16, (_P, _A), 1)).astype(jnp.bfloat16)

    g = jnp.dot(oh, tbl[...],
                preferred_element_type=jnp.float32)
    x_j = g[:, :_D]
    r_j = g[:, _D:_D + 8] + g[:, _D + 8:_TW]

    ohc = ohc_ref[...]
    ctr = (tbl[pl.ds(t * _TA, _TA), _D:_D + 8].astype(jnp.float32)
           + tbl[pl.ds(t * _TA, _TA), _D + 8:_TW].astype(jnp.float32))
    r_i = jnp.dot(ohc, ctr, preferred_element_type=jnp.float32)
    diff = r_j - r_i
    d2 = jnp.sum(diff * diff, axis=1, keepdims=True)
    d = jnp.sqrt(d2 + 1e-8)

    centers = jax.lax.broadcasted_iota(
        jnp.int32, (1, _NRBF), 1).astype(jnp.float32) * (5.0 / (_NRBF - 1))
    delta = d - centers
    rbf = jnp.exp(-10.0 * delta * delta)
    wij = jnp.dot(rbf, wf1_ref[...],
                  preferred_element_type=jnp.float32)

    prod = x_j * wij
    msg = jax.lax.dot_general(ohc, prod, (((0,), (0,)), ((), ())),
                              preferred_element_type=jnp.float32)

    rep = (x_f32[pl.ds(t * _TA, _TA), :]
           + jnp.tanh(jnp.dot(msg, wmsg_ref[...],
                              preferred_element_type=jnp.float32)
                      + bmsg_ref[...]))
    h1 = jnp.tanh(jnp.dot(rep, wo1_ref[...],
                          preferred_element_type=jnp.float32)
                  + bo1_ref[...])
    atom_e = jnp.dot(h1, wo2_ref[...],
                     preferred_element_type=jnp.float32)
    te = jnp.sum(atom_e)

    @pl.when(t == 0)
    def _first():
        out_ref[...] = jnp.full((1, 1, 128), te, jnp.float32)

    @pl.when(t != 0)
    def _acc():
        out_ref[...] += jnp.full((1, 1, 128), te, jnp.float32)


def kernel(positions, atomic_numbers, neighbors, emb, W_f1, W_msg, b_msg,
           W_o1, b_o1, W_o2, b_o2):
    z3 = atomic_numbers.astype(jnp.int32).reshape(_B, 1, _A)
    nbr = neighbors.astype(jnp.int32).reshape(_B * _NT, _P, 1)
    ohc = jnp.asarray(np.kron(np.eye(_TA, dtype=np.float32),
                              np.ones((_N, 1), np.float32)))
    grid = (_B, _NT)
    out = pl.pallas_call(
        _body,
        grid=grid,
        in_specs=[
            pl.BlockSpec((1, _P, 1), lambda b, t: (b * _NT + t, 0, 0)),
            pl.BlockSpec((_P, _TA), lambda b, t: (0, 0)),
            pl.BlockSpec((1, _A, 3), lambda b, t: (b, 0, 0)),
            pl.BlockSpec((1, 1, _A), lambda b, t: (b, 0, 0)),
            pl.BlockSpec((_ZMAX, _D), lambda b, t: (0, 0)),
            pl.BlockSpec((_NRBF, _D), lambda b, t: (0, 0)),
            pl.BlockSpec((_D, _D), lambda b, t: (0, 0)),
            pl.BlockSpec((1, _D), lambda b, t: (0, 0)),
            pl.BlockSpec((_D, _H), lambda b, t: (0, 0)),
            pl.BlockSpec((1, _H), lambda b, t: (0, 0)),
            pl.BlockSpec((_H, 1), lambda b, t: (0, 0)),
        ],
        out_specs=pl.BlockSpec((1, 1, 128), lambda b, t: (b, 0, 0)),
        out_shape=jax.ShapeDtypeStruct((_B, 1, 128), jnp.float32),
        scratch_shapes=[
            pltpu.VMEM((_A, _D), jnp.float32),
            pltpu.VMEM((_A, _TW), jnp.bfloat16),
        ],
    )(nbr, ohc, positions, z3, emb, W_f1, W_msg, b_msg.reshape(1, _D),
      W_o1, b_o1.reshape(1, _H), W_o2)
    return out[:, 0, :1] + _A * b_o2[0]


# speedup vs baseline: 49.6255x; 6.3125x over previous
import functools

import jax
import jax.numpy as jnp
import numpy as np
from jax import lax
from jax.experimental import pallas as pl
from jax.experimental.pallas import tpu as pltpu
from jax.experimental.pallas import tpu_sc as plsc

_B, _A, _N, _D, _NRBF, _ZMAX, _H = 8, 1024, 48, 128, 32, 100, 64
_E = _B * _A * _N
_NW = 32
_EPT = _E // _NW
_ROWS = _EPT // 128
_CPB = _NW // _B

_TA = 128
_NT = _A // _TA
_P = _TA * _N
_ST = _B * _NT


def _sc_body(idx_hbm, ctr_hbm, z_hbm, px_hbm, py_hbm, pz_hbm, zj_out, d2_out,
             idx_v, ctr_v, zj_v, d2_v, z_v, px_v, py_v, pz_v):
    c = lax.axis_index("c")
    s = lax.axis_index("s")
    w = s * 2 + c
    b = w // _CPB

    pltpu.sync_copy(idx_hbm.at[w], idx_v)
    pltpu.sync_copy(ctr_hbm.at[w], ctr_v)
    pltpu.sync_copy(z_hbm.at[b], z_v)
    pltpu.sync_copy(px_hbm.at[b], px_v)
    pltpu.sync_copy(py_hbm.at[b], py_v)
    pltpu.sync_copy(pz_hbm.at[b], pz_v)

    def body(i, carry):
        row = i // 8
        col = (i % 8) * 16
        iv = idx_v[row, pl.ds(col, 16)]
        zj_v[row, pl.ds(col, 16)] = plsc.load_gather(z_v, [iv])
        xg = plsc.load_gather(px_v, [iv])
        yg = plsc.load_gather(py_v, [iv])
        zg = plsc.load_gather(pz_v, [iv])
        aidx = ctr_v[row, pl.ds(col, 16)]
        cx = plsc.load_gather(px_v, [aidx])
        cy = plsc.load_gather(py_v, [aidx])
        cz = plsc.load_gather(pz_v, [aidx])
        dx = xg - cx
        dy = yg - cy
        dz = zg - cz
        d2_v[row, pl.ds(col, 16)] = dx * dx + dy * dy + dz * dz
        return carry

    lax.fori_loop(0, _EPT // 16, body, 0)

    pltpu.sync_copy(zj_v, zj_out.at[w])
    pltpu.sync_copy(d2_v, d2_out.at[w])


def _sc_edges(idx, ctr, z, px, py, pz):
    mesh = plsc.VectorSubcoreMesh(core_axis_name="c", subcore_axis_name="s")
    f = functools.partial(
        pl.kernel, mesh=mesh,
        compiler_params=pltpu.CompilerParams(needs_layout_passes=False),
        out_type=[
            jax.ShapeDtypeStruct((_NW, _ROWS, 128), jnp.int32),
            jax.ShapeDtypeStruct((_NW, _ROWS, 128), jnp.float32),
        ],
        scratch_types=[
            pltpu.VMEM((_ROWS, 128), jnp.int32),
            pltpu.VMEM((_ROWS, 128), jnp.int32),
            pltpu.VMEM((_ROWS, 128), jnp.int32),
            pltpu.VMEM((_ROWS, 128), jnp.float32),
            pltpu.VMEM((_A,), jnp.int32),
            pltpu.VMEM((_A,), jnp.float32),
            pltpu.VMEM((_A,), jnp.float32),
            pltpu.VMEM((_A,), jnp.float32),
        ],
    )(_sc_body)
    return f(idx, ctr, z, px, py, pz)


def _tc_body(zj_ref, d2_ref, z_ref, embT_ref, wf1T_ref, ohc_ref, wmsgT_ref,
             bmsg_ref, wo1T_ref, bo1_ref, wo2T_ref, out_ref,
             xT, embTbf):
    t = pl.program_id(1)

    @pl.when(t == 0)
    def _init():
        z_row = z_ref[0, 0][None, :]
        ohzc = (z_row == lax.broadcasted_iota(
            jnp.int32, (_ZMAX, _A), 0)).astype(jnp.float32)
        xT[...] = jnp.dot(embT_ref[...], ohzc,
                          preferred_element_type=jnp.float32)
        embTbf[...] = embT_ref[...].astype(jnp.bfloat16)

    zrow = zj_ref[0]
    ohz = (zrow == lax.broadcasted_iota(
        jnp.int32, (_ZMAX, _P), 0)).astype(jnp.bfloat16)
    xjT = jnp.dot(embTbf[...], ohz,
                  preferred_element_type=jnp.float32)

    d = jnp.sqrt(d2_ref[0] + 1e-8)
    centers = lax.broadcasted_iota(
        jnp.int32, (_NRBF, 1), 0).astype(jnp.float32) * (5.0 / (_NRBF - 1))
    delta = d - centers
    rbfT = jnp.exp(-10.0 * delta * delta)
    wijT = jnp.dot(wf1T_ref[...], rbfT,
                   preferred_element_type=jnp.float32)

    prodT = (xjT * wijT).astype(jnp.bfloat16)
    msgT = jnp.dot(prodT, ohc_ref[...],
                   preferred_element_type=jnp.float32)

    repT = (xT[:, pl.ds(t * _TA, _TA)]
            + jnp.tanh(jnp.dot(wmsgT_ref[...], msgT,
                               preferred_element_type=jnp.float32)
                       + bmsg_ref[...]))
    h1T = jnp.tanh(jnp.dot(wo1T_ref[...], repT,
                           preferred_element_type=jnp.float32)
                   + bo1_ref[...])
    atom_eT = jnp.dot(wo2T_ref[...], h1T,
                      preferred_element_type=jnp.float32)
    te = jnp.sum(atom_eT)

    @pl.when(t == 0)
    def _first():
        out_ref[...] = jnp.full((1, 1, 128), te, jnp.float32)

    @pl.when(t != 0)
    def _acc():
        out_ref[...] += jnp.full((1, 1, 128), te, jnp.float32)


def kernel(positions, atomic_numbers, neighbors, emb, W_f1, W_msg, b_msg,
           W_o1, b_o1, W_o2, b_o2):
    z = atomic_numbers.astype(jnp.int32)
    idx = neighbors.astype(jnp.int32).reshape(_NW, _ROWS, 128)
    px = positions[:, :, 0]
    py = positions[:, :, 1]
    pz = positions[:, :, 2]

    ctr = jnp.asarray(
        (np.arange(_E, dtype=np.int32) // _N) % _A).reshape(_NW, _ROWS, 128)
    zj, d2 = _sc_edges(idx, ctr, z, px, py, pz)
    zj = zj.reshape(_ST, 1, _P)
    d2 = d2.reshape(_ST, 1, _P)

    ohc = jnp.asarray(np.kron(np.eye(_TA, dtype=np.float32),
                              np.ones((1, _N), np.float32)).T
                      ).astype(jnp.bfloat16)

    grid = (_B, _NT)
    out = pl.pallas_call(
        _tc_body,
        grid=grid,
        in_specs=[
            pl.BlockSpec((1, 1, _P), lambda b, t: (b * _NT + t, 0, 0)),
            pl.BlockSpec((1, 1, _P), lambda b, t: (b * _NT + t, 0, 0)),
            pl.BlockSpec((1, 1, _A), lambda b, t: (b, 0, 0)),
            pl.BlockSpec((_D, _ZMAX), lambda b, t: (0, 0)),
            pl.BlockSpec((_D, _NRBF), lambda b, t: (0, 0)),
            pl.BlockSpec((_P, _TA), lambda b, t: (0, 0)),
            pl.BlockSpec((_D, _D), lambda b, t: (0, 0)),
            pl.BlockSpec((_D, 1), lambda b, t: (0, 0)),
            pl.BlockSpec((_H, _D), lambda b, t: (0, 0)),
            pl.BlockSpec((_H, 1), lambda b, t: (0, 0)),
            pl.BlockSpec((1, _H), lambda b, t: (0, 0)),
        ],
        out_specs=pl.BlockSpec((1, 1, 128), lambda b, t: (b, 0, 0)),
        out_shape=jax.ShapeDtypeStruct((_B, 1, 128), jnp.float32),
        scratch_shapes=[
            pltpu.VMEM((_D, _A), jnp.float32),
            pltpu.VMEM((_D, _ZMAX), jnp.bfloat16),
        ],
    )(zj, d2, z.reshape(_B, 1, _A), emb.T, W_f1.T, ohc, W_msg.T,
      b_msg.reshape(_D, 1),
      W_o1.T, b_o1.reshape(_H, 1), W_o2.T)
    return out[:, 0, :1] + _A * b_o2[0]
